# Initial kernel scaffold; baseline (speedup 1.0000x reference)
import jax
import jax.numpy as jnp
from jax.experimental import pallas as pl


def kernel(x, gate_W, gate_b, A_all, B_all, gamma, beta):
    raise NotImplementedError("write your pallas kernel here")


# speedup vs baseline: 1.1896x; 1.1896x over previous
import functools

import jax
import jax.numpy as jnp
from jax import lax
from jax.experimental import pallas as pl
from jax.experimental.pallas import tpu as pltpu

_E = 8
_R = 8
_NEG = -3.0e38


def _mean_kernel(x_ref, h_ref, *, inv_s):
    s = pl.program_id(1)
    ns = pl.num_programs(1)

    @pl.when(s == 0)
    def _():
        h_ref[...] = jnp.zeros_like(h_ref)

    h_ref[...] += jnp.sum(x_ref[0], axis=0)[None, None, :]

    @pl.when(s == ns - 1)
    def _():
        h_ref[...] = h_ref[...] * inv_s


def _moe_ln_kernel(x_ref, h_ref, gw_ref, gb_ref, a2_ref, b2_ref,
                   gamma_ref, beta_ref, o_ref, delta_ref):
    s = pl.program_id(1)

    @pl.when(s == 0)
    def _():
        h = h_ref[0]
        logits = jnp.sum(gw_ref[...] * h, axis=1, keepdims=True) + gb_ref[...]
        iota8 = lax.broadcasted_iota(jnp.int32, (_E, 1), 0)
        m1 = jnp.max(logits)
        i1 = jnp.min(jnp.where(logits == m1, iota8, _E))
        masked = jnp.where(iota8 == i1, _NEG, logits)
        m2 = jnp.max(masked)
        i2 = jnp.min(jnp.where(masked == m2, iota8, _E))
        eb = jnp.exp(m2 - m1)
        denom = 1.0 + eb
        w1 = 1.0 / denom
        w2 = eb / denom
        t = jnp.sum(a2_ref[...] * h, axis=1, keepdims=True)
        e_ids = lax.broadcasted_iota(jnp.int32, (_E * _R, 1), 0) // _R
        wfull = (jnp.where(e_ids == i1, w1, 0.0)
                 + jnp.where(e_ids == i2, w2, 0.0))
        wt = wfull * t * (1.0 / _R)
        delta_ref[...] = jnp.sum(wt * b2_ref[...], axis=0, keepdims=True)

    y = x_ref[...] + delta_ref[...][:, None, :]
    mu = jnp.mean(y, axis=2, keepdims=True)
    yc = y - mu
    var = jnp.mean(yc * yc, axis=2, keepdims=True)
    o_ref[...] = (yc * lax.rsqrt(var + 1e-5) * gamma_ref[...][:, None, :]
                  + beta_ref[...][:, None, :])


def kernel(x, gate_W, gate_b, A_all, B_all, gamma, beta):
    B, S, D = x.shape
    s_blk = 256
    ns = S // s_blk

    h = pl.pallas_call(
        functools.partial(_mean_kernel, inv_s=1.0 / S),
        grid=(B, ns),
        in_specs=[pl.BlockSpec((1, s_blk, D), lambda b, s: (b, s, 0))],
        out_specs=pl.BlockSpec((1, 1, D), lambda b, s: (b, 0, 0)),
        out_shape=jax.ShapeDtypeStruct((B, 1, D), jnp.float32),
    )(x)

    A2 = A_all.reshape(_E * _R, D)
    B2 = jnp.transpose(B_all, (0, 2, 1)).reshape(_E * _R, D)
    gb = gate_b.reshape(_E, 1)
    gm = gamma.reshape(1, D)
    bt = beta.reshape(1, D)

    out = pl.pallas_call(
        _moe_ln_kernel,
        grid=(B, ns),
        in_specs=[
            pl.BlockSpec((1, s_blk, D), lambda b, s: (b, s, 0)),
            pl.BlockSpec((1, 1, D), lambda b, s: (b, 0, 0)),
            pl.BlockSpec((_E, D), lambda b, s: (0, 0)),
            pl.BlockSpec((_E, 1), lambda b, s: (0, 0)),
            pl.BlockSpec((_E * _R, D), lambda b, s: (0, 0)),
            pl.BlockSpec((_E * _R, D), lambda b, s: (0, 0)),
            pl.BlockSpec((1, D), lambda b, s: (0, 0)),
            pl.BlockSpec((1, D), lambda b, s: (0, 0)),
        ],
        out_specs=pl.BlockSpec((1, s_blk, D), lambda b, s: (b, s, 0)),
        out_shape=jax.ShapeDtypeStruct((B, S, D), jnp.float32),
        scratch_shapes=[pltpu.VMEM((1, D), jnp.float32)],
    )(x, h, gate_W, gb, A2, B2, gm, bt)
    return out
